# baseline (speedup 1.0000x reference)
import functools

import jax
import jax.numpy as jnp
from jax import lax
from jax.experimental import pallas as pl
from jax.experimental.pallas import tpu as pltpu
from jax.experimental.pallas import tpu_sc as plsc

N = 10000
D = 128
E = 320000

NC = 2
NS = 16
L = 16

CH = 64
CHUNKS = -(-E // CH)
K = -(-CHUNKS // (NC * NS * 8)) * 8
CHUNKS_PAD = K * NC * NS
EP = CHUNKS_PAD * CH
CPS = CHUNKS_PAD // NC
KB = K // 4
NSLOT = 4
K0 = 240
K1 = 2 * K - K0

ACC_ROWS = 10112
ZROWS = ACC_ROWS // NS
DEG_ROWS = 10240

_mesh = plsc.VectorSubcoreMesh(core_axis_name="c", subcore_axis_name="s")


def _zero_buf(buf, rows):
    def body(r, _):
        for cc in range(D // L):
            buf[r, pl.ds(cc * L, L)] = jnp.zeros((L,), jnp.float32)
        return 0
    lax.fori_loop(0, rows, body, 0)


@functools.partial(
    pl.kernel,
    out_type=jax.ShapeDtypeStruct((NC * N,), jnp.float32),
    mesh=_mesh,
    scratch_types=[
        pltpu.VMEM((K, CH), jnp.int32),
        pltpu.VMEM((CH,), jnp.float32),
        pltpu.VMEM((640,), jnp.float32),
        pltpu.VMEM((1000,), jnp.float32),
        pltpu.VMEM_SHARED((DEG_ROWS,), jnp.float32),
        pltpu.SemaphoreType.DMA,
    ],
)
def _deg_kernel(dst_hbm, out_hbm, dstbuf, ones, zbuf, relay, acc, sem):
    c = lax.axis_index("c")
    t = lax.axis_index("s")

    def fill(r, _):
        zbuf[pl.ds(r * L, L)] = jnp.zeros((L,), jnp.float32)
        return 0
    lax.fori_loop(0, 640 // L, fill, 0)
    for cc in range(CH // L):
        ones[pl.ds(cc * L, L)] = jnp.ones((L,), jnp.float32)
    pltpu.sync_copy(zbuf, acc.at[pl.ds(t * 640, 640)])
    plsc.subcore_barrier()

    base = c * CPS + t * K
    pltpu.sync_copy(dst_hbm.at[pl.ds(base, K)], dstbuf)

    def body(k, _):
        pltpu.async_copy(ones, acc.at[dstbuf.at[k]], sem, add=True)
        return 0
    lax.fori_loop(0, K, body, 0)

    def drain(k, _):
        pltpu.make_async_copy(ones, acc.at[dstbuf.at[0]], sem).wait()
        return 0
    lax.fori_loop(0, K, drain, 0)
    plsc.subcore_barrier()

    @pl.when(t < 10)
    def _():
        pltpu.sync_copy(acc.at[pl.ds(t * 1000, 1000)], relay)
        pltpu.sync_copy(relay, out_hbm.at[pl.ds(c * N + t * 1000, 1000)])


@functools.partial(
    pl.kernel,
    out_type=jax.ShapeDtypeStruct((NC, N, D), jnp.float32),
    mesh=_mesh,
    scratch_types=[
        pltpu.VMEM((KB, CH), jnp.int32),
        pltpu.VMEM((KB, CH), jnp.int32),
        pltpu.VMEM((NSLOT, CH, D), jnp.float32),
        pltpu.VMEM_SHARED((ACC_ROWS, D), jnp.float32),
    ] + [pltpu.SemaphoreType.DMA] * (2 * NSLOT),
)
def _edge_kernel(ga_hbm, gb_hbm, src_hbm, dst_hbm, out_hbm, srcbuf, dstbuf,
                 bufs, acc, *sems):
    c = lax.axis_index("c")
    t = lax.axis_index("s")
    gsems = sems[:NSLOT]
    ssems = sems[NSLOT:]
    kc = jnp.where(c == 0, K0, K1)
    base = jnp.where(c == 0, t * K0, K0 * NS + t * K1)

    _zero_buf(bufs.at[0], CH)
    for j in range(ZROWS // CH):
        pltpu.sync_copy(bufs.at[0], acc.at[pl.ds(t * ZROWS + j * CH, CH)])
    rem = ZROWS % CH
    if rem:
        pltpu.sync_copy(bufs.at[0].at[pl.ds(0, rem)],
                        acc.at[pl.ds(t * ZROWS + (ZROWS // CH) * CH, rem)])
    plsc.subcore_barrier()

    def run_edges(g_hbm):
        def block(h, _):
            off = pl.multiple_of(base + h * KB, 8)
            pltpu.sync_copy(src_hbm.at[pl.ds(off, KB)], srcbuf)
            pltpu.sync_copy(dst_hbm.at[pl.ds(off, KB)], dstbuf)
            for j in range(NSLOT):
                pltpu.async_copy(g_hbm.at[srcbuf.at[j]], bufs.at[j],
                                 gsems[j])

            def body(kk, _):
                for j in range(NSLOT):
                    k = kk * NSLOT + j
                    pltpu.make_async_copy(g_hbm.at[srcbuf.at[k]], bufs.at[j],
                                          gsems[j]).wait()
                    pltpu.async_copy(bufs.at[j], acc.at[dstbuf.at[k]],
                                     ssems[j], add=True)
                for j in range(NSLOT):
                    k = kk * NSLOT + j
                    pltpu.make_async_copy(bufs.at[j], acc.at[dstbuf.at[k]],
                                          ssems[j]).wait()

                    @pl.when(k + NSLOT < KB)
                    def _():
                        pltpu.async_copy(g_hbm.at[srcbuf.at[k + NSLOT]],
                                         bufs.at[j], gsems[j])
                return 0
            lax.fori_loop(0, KB // NSLOT, body, 0)
            return 0
        lax.fori_loop(0, kc // KB, block, 0)

    @pl.when(c == 0)
    def _():
        run_edges(ga_hbm)

    @pl.when(c == 1)
    def _():
        run_edges(gb_hbm)
    plsc.subcore_barrier()

    @pl.when(t < 10)
    def _():
        NOUT = 1000 // CH + 1

        def _out_desc(j):
            rows = CH if j < NOUT - 1 else 1000 - (NOUT - 1) * CH
            r0 = t * 1000 + j * CH
            return (bufs.at[j % NSLOT].at[pl.ds(0, rows)],
                    out_hbm.at[c, pl.ds(r0, rows)], ssems[j % NSLOT])
        for j in range(NOUT):
            rows = CH if j < NOUT - 1 else 1000 - (NOUT - 1) * CH
            r0 = t * 1000 + j * CH
            if j >= NSLOT:
                pltpu.make_async_copy(*_out_desc(j - NSLOT)).wait()
            pltpu.async_copy(acc.at[pl.ds(r0, rows)],
                             bufs.at[j % NSLOT].at[pl.ds(0, rows)],
                             gsems[j % NSLOT]).wait()
            pltpu.async_copy(*_out_desc(j))
        for j in range(NOUT - NSLOT, NOUT):
            pltpu.make_async_copy(*_out_desc(j)).wait()


def _tc_call(body, out_shapes, in_specs, out_specs, grid):
    return pl.pallas_call(
        body,
        grid=grid,
        in_specs=in_specs,
        out_specs=out_specs,
        out_shape=out_shapes,
    )


BR = 1000


def _spec_rows():
    return pl.BlockSpec((BR, D), lambda i: (i, 0))


def _spec_col():
    return pl.BlockSpec((BR, 1), lambda i: (i, 0))


def _spec_w():
    return pl.BlockSpec((D, D), lambda i: (0, 0))


def _spec_b():
    return pl.BlockSpec((1, D), lambda i: (0, 0))


def _spec_p():
    return pl.BlockSpec((NC, BR, D), lambda i: (0, i, 0))


def _tc_a_body(x_ref, w1_ref, degp_ref, dinv_ref, g1_ref, g1b_ref):
    deg = degp_ref[0] + degp_ref[1] + 1.0
    dinv = lax.rsqrt(deg)
    dinv_ref[...] = dinv
    h = jnp.dot(x_ref[...], w1_ref[...], preferred_element_type=jnp.float32)
    g1 = h * dinv
    g1_ref[...] = g1
    g1b_ref[...] = g1


def _tc_b_body(p_ref, g1_ref, b1_ref, dinv_ref, w2_ref, h1_ref, g2_ref,
               g2b_ref):
    dinv = dinv_ref[...]
    conv = dinv * (p_ref[0] + p_ref[1] + g1_ref[...]) + b1_ref[...]
    h1 = jnp.maximum(conv, 0.0)
    h1_ref[...] = h1
    g2 = jnp.dot(h1, w2_ref[...], preferred_element_type=jnp.float32) * dinv
    g2_ref[...] = g2
    g2b_ref[...] = g2


def _tc_c_body(p_ref, g2_ref, b2_ref, dinv_ref, h1_ref, w3_ref, g3_ref,
               g3b_ref):
    dinv = dinv_ref[...]
    conv = dinv * (p_ref[0] + p_ref[1] + g2_ref[...]) + b2_ref[...]
    h2 = h1_ref[...] + jnp.maximum(conv, 0.0)
    g3 = jnp.dot(h2, w3_ref[...], preferred_element_type=jnp.float32) * dinv
    g3_ref[...] = g3
    g3b_ref[...] = g3


def _tc_d_body(p_ref, g3_ref, b3_ref, dinv_ref, out_ref):
    out_ref[...] = (dinv_ref[...] * (p_ref[0] + p_ref[1] + g3_ref[...])
                    + b3_ref[...])


def kernel(x, edge_index, edge_weights, W1, b1, W2, b2, W3, b3):
    del edge_weights
    src = edge_index[0].astype(jnp.int32)
    dst = edge_index[1].astype(jnp.int32)
    pad = EP - E
    src_p = jnp.concatenate([src, jnp.zeros((pad,), jnp.int32)]).reshape(
        CHUNKS_PAD, CH)
    dst_p = jnp.concatenate([dst, jnp.full((pad,), N, jnp.int32)]).reshape(
        CHUNKS_PAD, CH)

    degp = _deg_kernel(dst_p)
    degp = degp.reshape(NC, N, 1)

    b1r = b1.reshape(1, D)
    b2r = b2.reshape(1, D)
    b3r = b3.reshape(1, D)

    grid = (N // BR,)
    dinv, g1, g1b = _tc_call(
        _tc_a_body,
        (jax.ShapeDtypeStruct((N, 1), jnp.float32),
         jax.ShapeDtypeStruct((N, D), jnp.float32),
         jax.ShapeDtypeStruct((N, D), jnp.float32)),
        [_spec_rows(), _spec_w(),
         pl.BlockSpec((NC, BR, 1), lambda i: (0, i, 0))],
        (_spec_col(), _spec_rows(), _spec_rows()),
        grid)(x, W1, degp)

    p1 = _edge_kernel(g1, g1b, src_p, dst_p)
    h1, g2, g2b = _tc_call(
        _tc_b_body,
        (jax.ShapeDtypeStruct((N, D), jnp.float32),
         jax.ShapeDtypeStruct((N, D), jnp.float32),
         jax.ShapeDtypeStruct((N, D), jnp.float32)),
        [_spec_p(), _spec_rows(), _spec_b(), _spec_col(), _spec_w()],
        (_spec_rows(), _spec_rows(), _spec_rows()),
        grid)(p1, g1, b1r, dinv, W2)

    p2 = _edge_kernel(g2, g2b, src_p, dst_p)
    g3, g3b = _tc_call(
        _tc_c_body,
        (jax.ShapeDtypeStruct((N, D), jnp.float32),
         jax.ShapeDtypeStruct((N, D), jnp.float32)),
        [_spec_p(), _spec_rows(), _spec_b(), _spec_col(), _spec_rows(),
         _spec_w()],
        (_spec_rows(), _spec_rows()),
        grid)(p2, g2, b2r, dinv, h1, W3)

    p3 = _edge_kernel(g3, g3b, src_p, dst_p)
    out, = _tc_call(
        _tc_d_body,
        (jax.ShapeDtypeStruct((N, D), jnp.float32),),
        [_spec_p(), _spec_rows(), _spec_b(), _spec_col()],
        (_spec_rows(),),
        grid)(p3, g3, b3r, dinv)
    return out


# speedup vs baseline: 1.0119x; 1.0119x over previous
import functools

import jax
import jax.numpy as jnp
from jax import lax
from jax.experimental import pallas as pl
from jax.experimental.pallas import tpu as pltpu
from jax.experimental.pallas import tpu_sc as plsc

N = 10000
D = 128
E = 320000

NC = 2
NS = 16
L = 16

CH = 64
CHUNKS = -(-E // CH)
K = -(-CHUNKS // (NC * NS * 8)) * 8
CHUNKS_PAD = K * NC * NS
EP = CHUNKS_PAD * CH
CPS = CHUNKS_PAD // NC
KB = K // 4
NSLOT = 4
K0 = 280
K1 = 2 * K - K0

ACC_ROWS = 10112
ZROWS = ACC_ROWS // NS
DEG_ROWS = 10240

_mesh = plsc.VectorSubcoreMesh(core_axis_name="c", subcore_axis_name="s")


def _zero_buf(buf, rows):
    def body(r, _):
        for cc in range(D // L):
            buf[r, pl.ds(cc * L, L)] = jnp.zeros((L,), jnp.float32)
        return 0
    lax.fori_loop(0, rows, body, 0)


@functools.partial(
    pl.kernel,
    out_type=jax.ShapeDtypeStruct((NC * N,), jnp.float32),
    mesh=_mesh,
    scratch_types=[
        pltpu.VMEM((K, CH), jnp.int32),
        pltpu.VMEM((CH,), jnp.float32),
        pltpu.VMEM((640,), jnp.float32),
        pltpu.VMEM((1000,), jnp.float32),
        pltpu.VMEM_SHARED((DEG_ROWS,), jnp.float32),
        pltpu.SemaphoreType.DMA,
    ],
)
def _deg_kernel(dst_hbm, out_hbm, dstbuf, ones, zbuf, relay, acc, sem):
    c = lax.axis_index("c")
    t = lax.axis_index("s")

    def fill(r, _):
        zbuf[pl.ds(r * L, L)] = jnp.zeros((L,), jnp.float32)
        return 0
    lax.fori_loop(0, 640 // L, fill, 0)
    for cc in range(CH // L):
        ones[pl.ds(cc * L, L)] = jnp.ones((L,), jnp.float32)
    pltpu.sync_copy(zbuf, acc.at[pl.ds(t * 640, 640)])
    plsc.subcore_barrier()

    base = c * CPS + t * K
    pltpu.sync_copy(dst_hbm.at[pl.ds(base, K)], dstbuf)

    def body(k, _):
        pltpu.async_copy(ones, acc.at[dstbuf.at[k]], sem, add=True)
        return 0
    lax.fori_loop(0, K, body, 0)

    def drain(k, _):
        pltpu.make_async_copy(ones, acc.at[dstbuf.at[0]], sem).wait()
        return 0
    lax.fori_loop(0, K, drain, 0)
    plsc.subcore_barrier()

    @pl.when(t < 10)
    def _():
        pltpu.sync_copy(acc.at[pl.ds(t * 1000, 1000)], relay)
        pltpu.sync_copy(relay, out_hbm.at[pl.ds(c * N + t * 1000, 1000)])


@functools.partial(
    pl.kernel,
    out_type=jax.ShapeDtypeStruct((NC, N, D), jnp.float32),
    mesh=_mesh,
    scratch_types=[
        pltpu.VMEM((KB, CH), jnp.int32),
        pltpu.VMEM((KB, CH), jnp.int32),
        pltpu.VMEM((NSLOT, CH, D), jnp.float32),
        pltpu.VMEM_SHARED((ACC_ROWS, D), jnp.float32),
    ] + [pltpu.SemaphoreType.DMA] * (2 * NSLOT),
)
def _edge_kernel(ga_hbm, gb_hbm, src_hbm, dst_hbm, out_hbm, srcbuf, dstbuf,
                 bufs, acc, *sems):
    c = lax.axis_index("c")
    t = lax.axis_index("s")
    gsems = sems[:NSLOT]
    ssems = sems[NSLOT:]
    kc = jnp.where(c == 0, K0, K1)
    base = jnp.where(c == 0, t * K0, K0 * NS + t * K1)

    _zero_buf(bufs.at[0], CH)
    nz = ZROWS // CH
    rem = ZROWS % CH

    def _z_desc(j):
        rows = CH if j < nz else rem
        return (bufs.at[0].at[pl.ds(0, rows)],
                acc.at[pl.ds(t * ZROWS + j * CH, rows)], ssems[j % NSLOT])
    for j in range(nz + (1 if rem else 0)):
        pltpu.async_copy(*_z_desc(j))
    for j in range(nz + (1 if rem else 0)):
        pltpu.make_async_copy(*_z_desc(j)).wait()
    plsc.subcore_barrier()

    def run_edges(g_hbm):
        def block(h, _):
            off = pl.multiple_of(base + h * KB, 8)
            pltpu.sync_copy(src_hbm.at[pl.ds(off, KB)], srcbuf)
            pltpu.sync_copy(dst_hbm.at[pl.ds(off, KB)], dstbuf)
            for j in range(NSLOT):
                pltpu.async_copy(g_hbm.at[srcbuf.at[j]], bufs.at[j],
                                 gsems[j])

            def body(kk, _):
                for j in range(NSLOT):
                    k = kk * NSLOT + j
                    pltpu.make_async_copy(g_hbm.at[srcbuf.at[k]], bufs.at[j],
                                          gsems[j]).wait()
                    pltpu.async_copy(bufs.at[j], acc.at[dstbuf.at[k]],
                                     ssems[j], add=True)
                for j in range(NSLOT):
                    k = kk * NSLOT + j
                    pltpu.make_async_copy(bufs.at[j], acc.at[dstbuf.at[k]],
                                          ssems[j]).wait()

                    @pl.when(k + NSLOT < KB)
                    def _():
                        pltpu.async_copy(g_hbm.at[srcbuf.at[k + NSLOT]],
                                         bufs.at[j], gsems[j])
                return 0
            lax.fori_loop(0, KB // NSLOT, body, 0)
            return 0
        lax.fori_loop(0, kc // KB, block, 0)

    @pl.when(c == 0)
    def _():
        run_edges(ga_hbm)

    @pl.when(c == 1)
    def _():
        run_edges(gb_hbm)
    plsc.subcore_barrier()

    def copy_out(r0, total):
        nout = -(-total // CH)

        def _out_desc(j):
            rows = min(CH, total - j * CH)
            return (bufs.at[j % NSLOT].at[pl.ds(0, rows)],
                    out_hbm.at[c, pl.ds(r0 + j * CH, rows)],
                    ssems[j % NSLOT])
        for j in range(nout):
            rows = min(CH, total - j * CH)
            if j >= NSLOT:
                pltpu.make_async_copy(*_out_desc(j - NSLOT)).wait()
            pltpu.async_copy(acc.at[pl.ds(r0 + j * CH, rows)],
                             bufs.at[j % NSLOT].at[pl.ds(0, rows)],
                             gsems[j % NSLOT]).wait()
            pltpu.async_copy(*_out_desc(j))
        for j in range(max(0, nout - NSLOT), nout):
            pltpu.make_async_copy(*_out_desc(j)).wait()

    @pl.when(t < NS - 1)
    def _():
        copy_out(t * ZROWS, ZROWS)

    @pl.when(t == NS - 1)
    def _():
        copy_out((NS - 1) * ZROWS, N - (NS - 1) * ZROWS)


def _tc_call(body, out_shapes, in_specs, out_specs, grid):
    return pl.pallas_call(
        body,
        grid=grid,
        in_specs=in_specs,
        out_specs=out_specs,
        out_shape=out_shapes,
    )


BR = 1000


def _spec_rows():
    return pl.BlockSpec((BR, D), lambda i: (i, 0))


def _spec_col():
    return pl.BlockSpec((BR, 1), lambda i: (i, 0))


def _spec_w():
    return pl.BlockSpec((D, D), lambda i: (0, 0))


def _spec_b():
    return pl.BlockSpec((1, D), lambda i: (0, 0))


def _spec_p():
    return pl.BlockSpec((NC, BR, D), lambda i: (0, i, 0))


def _tc_mm_body(x_ref, w1_ref, xw_ref):
    xw_ref[...] = jnp.dot(x_ref[...], w1_ref[...],
                          preferred_element_type=jnp.float32)


def _tc_a_body(xw_ref, degp_ref, dinv_ref, g1_ref, g1b_ref):
    deg = degp_ref[0] + degp_ref[1] + 1.0
    dinv = lax.rsqrt(deg)
    dinv_ref[...] = dinv
    g1 = xw_ref[...] * dinv
    g1_ref[...] = g1
    g1b_ref[...] = g1


def _tc_b_body(p_ref, g1_ref, b1_ref, dinv_ref, w2_ref, h1_ref, g2_ref,
               g2b_ref):
    dinv = dinv_ref[...]
    conv = dinv * (p_ref[0] + p_ref[1] + g1_ref[...]) + b1_ref[...]
    h1 = jnp.maximum(conv, 0.0)
    h1_ref[...] = h1
    g2 = jnp.dot(h1, w2_ref[...], preferred_element_type=jnp.float32) * dinv
    g2_ref[...] = g2
    g2b_ref[...] = g2


def _tc_c_body(p_ref, g2_ref, b2_ref, dinv_ref, h1_ref, w3_ref, g3_ref,
               g3b_ref):
    dinv = dinv_ref[...]
    conv = dinv * (p_ref[0] + p_ref[1] + g2_ref[...]) + b2_ref[...]
    h2 = h1_ref[...] + jnp.maximum(conv, 0.0)
    g3 = jnp.dot(h2, w3_ref[...], preferred_element_type=jnp.float32) * dinv
    g3_ref[...] = g3
    g3b_ref[...] = g3


def _tc_d_body(p_ref, g3_ref, b3_ref, dinv_ref, out_ref):
    out_ref[...] = (dinv_ref[...] * (p_ref[0] + p_ref[1] + g3_ref[...])
                    + b3_ref[...])


def kernel(x, edge_index, edge_weights, W1, b1, W2, b2, W3, b3):
    del edge_weights
    src = edge_index[0].astype(jnp.int32)
    dst = edge_index[1].astype(jnp.int32)
    pad = EP - E
    src_p = jnp.concatenate([src, jnp.zeros((pad,), jnp.int32)]).reshape(
        CHUNKS_PAD, CH)
    dst_p = jnp.concatenate([dst, jnp.full((pad,), N, jnp.int32)]).reshape(
        CHUNKS_PAD, CH)

    degp = _deg_kernel(dst_p)
    degp = degp.reshape(NC, N, 1)
    xw, = _tc_call(
        _tc_mm_body,
        (jax.ShapeDtypeStruct((N, D), jnp.float32),),
        [_spec_rows(), _spec_w()],
        (_spec_rows(),),
        (N // BR,))(x, W1)

    b1r = b1.reshape(1, D)
    b2r = b2.reshape(1, D)
    b3r = b3.reshape(1, D)

    grid = (N // BR,)
    dinv, g1, g1b = _tc_call(
        _tc_a_body,
        (jax.ShapeDtypeStruct((N, 1), jnp.float32),
         jax.ShapeDtypeStruct((N, D), jnp.float32),
         jax.ShapeDtypeStruct((N, D), jnp.float32)),
        [_spec_rows(),
         pl.BlockSpec((NC, BR, 1), lambda i: (0, i, 0))],
        (_spec_col(), _spec_rows(), _spec_rows()),
        grid)(xw, degp)

    p1 = _edge_kernel(g1, g1b, src_p, dst_p)
    h1, g2, g2b = _tc_call(
        _tc_b_body,
        (jax.ShapeDtypeStruct((N, D), jnp.float32),
         jax.ShapeDtypeStruct((N, D), jnp.float32),
         jax.ShapeDtypeStruct((N, D), jnp.float32)),
        [_spec_p(), _spec_rows(), _spec_b(), _spec_col(), _spec_w()],
        (_spec_rows(), _spec_rows(), _spec_rows()),
        grid)(p1, g1, b1r, dinv, W2)

    p2 = _edge_kernel(g2, g2b, src_p, dst_p)
    g3, g3b = _tc_call(
        _tc_c_body,
        (jax.ShapeDtypeStruct((N, D), jnp.float32),
         jax.ShapeDtypeStruct((N, D), jnp.float32)),
        [_spec_p(), _spec_rows(), _spec_b(), _spec_col(), _spec_rows(),
         _spec_w()],
        (_spec_rows(), _spec_rows()),
        grid)(p2, g2, b2r, dinv, h1, W3)

    p3 = _edge_kernel(g3, g3b, src_p, dst_p)
    out, = _tc_call(
        _tc_d_body,
        (jax.ShapeDtypeStruct((N, D), jnp.float32),),
        [_spec_p(), _spec_rows(), _spec_b(), _spec_col()],
        (_spec_rows(),),
        grid)(p3, g3, b3r, dinv)
    return out


# speedup vs baseline: 1.0380x; 1.0258x over previous
import functools

import jax
import jax.numpy as jnp
from jax import lax
from jax.experimental import pallas as pl
from jax.experimental.pallas import tpu as pltpu
from jax.experimental.pallas import tpu_sc as plsc

N = 10000
D = 128
E = 320000

NC = 2
NS = 16
L = 16

CH = 64
CHUNKS = -(-E // CH)
K = -(-CHUNKS // (NC * NS * 8)) * 8
CHUNKS_PAD = K * NC * NS
EP = CHUNKS_PAD * CH
CPS = CHUNKS_PAD // NC
KB = K // 4
NSLOT = 4
K0 = 280
K1 = 2 * K - K0

ACC_ROWS = 10112
ZROWS = ACC_ROWS // NS
DEG_ROWS = 10240

_mesh = plsc.VectorSubcoreMesh(core_axis_name="c", subcore_axis_name="s")


def _zero_buf(buf, rows):
    def body(r, _):
        for cc in range(D // L):
            buf[r, pl.ds(cc * L, L)] = jnp.zeros((L,), jnp.float32)
        return 0
    lax.fori_loop(0, rows, body, 0)


@functools.partial(
    pl.kernel,
    out_type=jax.ShapeDtypeStruct((NC * N,), jnp.float32),
    mesh=_mesh,
    scratch_types=[
        pltpu.VMEM((K, CH), jnp.int32),
        pltpu.VMEM((CH,), jnp.float32),
        pltpu.VMEM((640,), jnp.float32),
        pltpu.VMEM((1000,), jnp.float32),
        pltpu.VMEM_SHARED((DEG_ROWS,), jnp.float32),
        pltpu.SemaphoreType.DMA,
    ],
)
def _deg_kernel(dst_hbm, out_hbm, dstbuf, ones, zbuf, relay, acc, sem):
    c = lax.axis_index("c")
    t = lax.axis_index("s")

    def fill(r, _):
        zbuf[pl.ds(r * L, L)] = jnp.zeros((L,), jnp.float32)
        return 0
    lax.fori_loop(0, 640 // L, fill, 0)
    for cc in range(CH // L):
        ones[pl.ds(cc * L, L)] = jnp.ones((L,), jnp.float32)
    pltpu.sync_copy(zbuf, acc.at[pl.ds(t * 640, 640)])
    plsc.subcore_barrier()

    base = c * CPS + t * K
    pltpu.sync_copy(dst_hbm.at[pl.ds(base, K)], dstbuf)

    def body(k, _):
        pltpu.async_copy(ones, acc.at[dstbuf.at[k]], sem, add=True)
        return 0
    lax.fori_loop(0, K, body, 0)

    def drain(k, _):
        pltpu.make_async_copy(ones, acc.at[dstbuf.at[0]], sem).wait()
        return 0
    lax.fori_loop(0, K, drain, 0)
    plsc.subcore_barrier()

    @pl.when(t < 10)
    def _():
        pltpu.sync_copy(acc.at[pl.ds(t * 1000, 1000)], relay)
        pltpu.sync_copy(relay, out_hbm.at[pl.ds(c * N + t * 1000, 1000)])


@functools.partial(
    pl.kernel,
    out_type=jax.ShapeDtypeStruct((NC, N, D), jnp.float32),
    mesh=_mesh,
    scratch_types=[
        pltpu.VMEM((KB, CH), jnp.int32),
        pltpu.VMEM((KB, CH), jnp.int32),
        pltpu.VMEM((NSLOT, CH, D), jnp.float32),
        pltpu.VMEM_SHARED((ACC_ROWS, D), jnp.float32),
    ] + [pltpu.SemaphoreType.DMA] * (2 * NSLOT),
)
def _edge_kernel(ga_hbm, gb_hbm, src_hbm, dst_hbm, out_hbm, srcbuf, dstbuf,
                 bufs, acc, *sems):
    c = lax.axis_index("c")
    t = lax.axis_index("s")
    gsems = sems[:NSLOT]
    ssems = sems[NSLOT:]
    kc = jnp.where(c == 0, K0, K1)
    base = jnp.where(c == 0, t * K0, K0 * NS + t * K1)

    _zero_buf(bufs.at[0], CH)
    nz = ZROWS // CH
    rem = ZROWS % CH

    def _z_desc(j):
        rows = CH if j < nz else rem
        return (bufs.at[0].at[pl.ds(0, rows)],
                acc.at[pl.ds(t * ZROWS + j * CH, rows)], ssems[j % NSLOT])
    for j in range(nz + (1 if rem else 0)):
        pltpu.async_copy(*_z_desc(j))
    for j in range(nz + (1 if rem else 0)):
        pltpu.make_async_copy(*_z_desc(j)).wait()
    plsc.subcore_barrier()

    def run_edges(g_hbm):
        def block(h, _):
            off = pl.multiple_of(base + h * KB, 8)
            pltpu.sync_copy(src_hbm.at[pl.ds(off, KB)], srcbuf)
            pltpu.sync_copy(dst_hbm.at[pl.ds(off, KB)], dstbuf)
            for j in range(NSLOT):
                pltpu.async_copy(g_hbm.at[srcbuf.at[j]], bufs.at[j],
                                 gsems[j])

            def body(kk, _):
                for j in range(NSLOT):
                    k = kk * NSLOT + j
                    pltpu.make_async_copy(g_hbm.at[srcbuf.at[k]], bufs.at[j],
                                          gsems[j]).wait()
                    pltpu.async_copy(bufs.at[j], acc.at[dstbuf.at[k]],
                                     ssems[j], add=True)
                for j in range(NSLOT):
                    k = kk * NSLOT + j
                    pltpu.make_async_copy(bufs.at[j], acc.at[dstbuf.at[k]],
                                          ssems[j]).wait()

                    @pl.when(k + NSLOT < KB)
                    def _():
                        pltpu.async_copy(g_hbm.at[srcbuf.at[k + NSLOT]],
                                         bufs.at[j], gsems[j])
                return 0
            lax.fori_loop(0, KB // NSLOT, body, 0)
            return 0
        lax.fori_loop(0, kc // KB, block, 0)

    @pl.when(c == 0)
    def _():
        run_edges(ga_hbm)

    @pl.when(c == 1)
    def _():
        run_edges(gb_hbm)
    plsc.subcore_barrier()

    def copy_out(r0, total):
        nout = -(-total // CH)

        def _out_desc(j):
            rows = min(CH, total - j * CH)
            return (bufs.at[j % NSLOT].at[pl.ds(0, rows)],
                    out_hbm.at[c, pl.ds(r0 + j * CH, rows)],
                    ssems[j % NSLOT])
        for j in range(nout):
            rows = min(CH, total - j * CH)
            if j >= NSLOT:
                pltpu.make_async_copy(*_out_desc(j - NSLOT)).wait()
            pltpu.async_copy(acc.at[pl.ds(r0 + j * CH, rows)],
                             bufs.at[j % NSLOT].at[pl.ds(0, rows)],
                             gsems[j % NSLOT]).wait()
            pltpu.async_copy(*_out_desc(j))
        for j in range(max(0, nout - NSLOT), nout):
            pltpu.make_async_copy(*_out_desc(j)).wait()

    @pl.when(t < NS - 1)
    def _():
        copy_out(t * ZROWS, ZROWS)

    @pl.when(t == NS - 1)
    def _():
        copy_out((NS - 1) * ZROWS, N - (NS - 1) * ZROWS)


def _tc_call(body, out_shapes, in_specs, out_specs, grid):
    return pl.pallas_call(
        body,
        grid=grid,
        in_specs=in_specs,
        out_specs=out_specs,
        out_shape=out_shapes,
    )


BR = 1000


def _spec_rows():
    return pl.BlockSpec((BR, D), lambda i: (i, 0))


def _spec_col():
    return pl.BlockSpec((BR, 1), lambda i: (i, 0))


def _spec_w():
    return pl.BlockSpec((D, D), lambda i: (0, 0))


def _spec_b():
    return pl.BlockSpec((1, D), lambda i: (0, 0))


def _spec_p():
    return pl.BlockSpec((NC, BR, D), lambda i: (0, i, 0))


def _tc_a_body(x_ref, w1_ref, degp_ref, dinv_ref, g1_ref, g1b_ref):
    deg = degp_ref[0] + degp_ref[1] + 1.0
    dinv = lax.rsqrt(deg)
    dinv_ref[...] = dinv
    h = jnp.dot(x_ref[...], w1_ref[...], preferred_element_type=jnp.float32)
    g1 = h * dinv
    g1_ref[...] = g1
    g1b_ref[...] = g1


def _tc_b_body(p_ref, g1_ref, b1_ref, dinv_ref, w2_ref, h1_ref, g2_ref,
               g2b_ref):
    dinv = dinv_ref[...]
    conv = dinv * (p_ref[0] + p_ref[1] + g1_ref[...]) + b1_ref[...]
    h1 = jnp.maximum(conv, 0.0)
    h1_ref[...] = h1
    g2 = jnp.dot(h1, w2_ref[...], preferred_element_type=jnp.float32) * dinv
    g2_ref[...] = g2
    g2b_ref[...] = g2


def _tc_c_body(p_ref, g2_ref, b2_ref, dinv_ref, h1_ref, w3_ref, g3_ref,
               g3b_ref):
    dinv = dinv_ref[...]
    conv = dinv * (p_ref[0] + p_ref[1] + g2_ref[...]) + b2_ref[...]
    h2 = h1_ref[...] + jnp.maximum(conv, 0.0)
    g3 = jnp.dot(h2, w3_ref[...], preferred_element_type=jnp.float32) * dinv
    g3_ref[...] = g3
    g3b_ref[...] = g3


def _tc_d_body(p_ref, g3_ref, b3_ref, dinv_ref, out_ref):
    out_ref[...] = (dinv_ref[...] * (p_ref[0] + p_ref[1] + g3_ref[...])
                    + b3_ref[...])


def kernel(x, edge_index, edge_weights, W1, b1, W2, b2, W3, b3):
    del edge_weights
    src = edge_index[0].astype(jnp.int32)
    dst = edge_index[1].astype(jnp.int32)
    pad = EP - E
    src_p = jnp.concatenate([src, jnp.zeros((pad,), jnp.int32)]).reshape(
        CHUNKS_PAD, CH)
    dst_p = jnp.concatenate([dst, jnp.full((pad,), N, jnp.int32)]).reshape(
        CHUNKS_PAD, CH)

    degp = _deg_kernel(dst_p)
    degp = degp.reshape(NC, N, 1)

    b1r = b1.reshape(1, D)
    b2r = b2.reshape(1, D)
    b3r = b3.reshape(1, D)

    grid = (N // BR,)
    dinv, g1, g1b = _tc_call(
        _tc_a_body,
        (jax.ShapeDtypeStruct((N, 1), jnp.float32),
         jax.ShapeDtypeStruct((N, D), jnp.float32),
         jax.ShapeDtypeStruct((N, D), jnp.float32)),
        [_spec_rows(), _spec_w(),
         pl.BlockSpec((NC, BR, 1), lambda i: (0, i, 0))],
        (_spec_col(), _spec_rows(), _spec_rows()),
        grid)(x, W1, degp)

    p1 = _edge_kernel(g1, g1b, src_p, dst_p)
    h1, g2, g2b = _tc_call(
        _tc_b_body,
        (jax.ShapeDtypeStruct((N, D), jnp.float32),
         jax.ShapeDtypeStruct((N, D), jnp.float32),
         jax.ShapeDtypeStruct((N, D), jnp.float32)),
        [_spec_p(), _spec_rows(), _spec_b(), _spec_col(), _spec_w()],
        (_spec_rows(), _spec_rows(), _spec_rows()),
        grid)(p1, g1, b1r, dinv, W2)

    p2 = _edge_kernel(g2, g2b, src_p, dst_p)
    g3, g3b = _tc_call(
        _tc_c_body,
        (jax.ShapeDtypeStruct((N, D), jnp.float32),
         jax.ShapeDtypeStruct((N, D), jnp.float32)),
        [_spec_p(), _spec_rows(), _spec_b(), _spec_col(), _spec_rows(),
         _spec_w()],
        (_spec_rows(), _spec_rows()),
        grid)(p2, g2, b2r, dinv, h1, W3)

    p3 = _edge_kernel(g3, g3b, src_p, dst_p)
    out, = _tc_call(
        _tc_d_body,
        (jax.ShapeDtypeStruct((N, D), jnp.float32),),
        [_spec_p(), _spec_rows(), _spec_b(), _spec_col()],
        (_spec_rows(),),
        grid)(p3, g3, b3r, dinv)
    return out
